# baseline (speedup 1.0000x reference)
import functools

import jax
import jax.numpy as jnp
from jax import lax
from jax.experimental import pallas as pl
from jax.experimental.pallas import tpu as pltpu
from jax.experimental.pallas import tpu_sc as plsc

_LANES = 16
_W = 256


def kernel(input_ids, table):
    B, S, H = input_ids.shape
    D = table.shape[-1]
    n_per_head = table.shape[0] // H
    pair_cols = 2 * D

    ids_bhs = jnp.transpose(input_ids, (0, 2, 1))
    table_pairs = table.reshape(table.shape[0] // 2, pair_cols)

    mesh = plsc.VectorSubcoreMesh(
        core_axis_name="core", subcore_axis_name="subcore"
    )
    n_windows = S // _W

    @functools.partial(
        pl.kernel,
        out_type=jax.ShapeDtypeStruct((B, H, D, S), table.dtype),
        mesh=mesh,
        scratch_types=[
            pltpu.VMEM((S,), jnp.int32),
            pltpu.VMEM((S,), jnp.int32),
            pltpu.VMEM((S,), jnp.int32),
            pltpu.VMEM((2, _W, pair_cols), jnp.float32),
            pltpu.VMEM((2, D, _W), jnp.float32),
            pltpu.SemaphoreType.DMA,
            pltpu.SemaphoreType.DMA,
        ],
        compiler_params=pltpu.CompilerParams(
            use_tc_tiling_on_sc=True, needs_layout_passes=False
        ),
    )
    def gather_kernel(ids_hbm, table_hbm, out_hbm, ids_v, hi_v, hcol_v,
                      bufa, buft, sem_g, sem_o):
        w = lax.axis_index("subcore") * 2 + lax.axis_index("core")
        b = w // H
        h = w % H
        off = h * n_per_head

        pltpu.sync_copy(ids_hbm.at[b, h], ids_v)

        @pl.loop(0, S, step=_LANES)
        def _(j):
            v = ids_v[pl.ds(j, _LANES)] + off
            hi_v[pl.ds(j, _LANES)] = lax.shift_right_logical(v, 1)
            hcol_v[pl.ds(j, _LANES)] = (v & 1) * D

        def start_gather(g, slot):
            return pltpu.async_copy(
                table_hbm.at[hi_v.at[pl.ds(g * _W, _W)]],
                bufa.at[slot],
                sem_g,
            )

        start_gather(0, 0)
        out_copies = [None, None]

        for g in range(n_windows):
            slot = g % 2
            pltpu.make_async_copy(
                table_hbm.at[hi_v.at[pl.ds(g * _W, _W)]],
                bufa.at[slot],
                sem_g,
            ).wait()
            if g + 1 < n_windows:
                start_gather(g + 1, 1 - slot)

            if out_copies[slot] is not None:
                out_copies[slot].wait()

            @pl.loop(0, _W, step=_LANES)
            def _(w0):
                rows16 = w0 + lax.iota(jnp.int32, _LANES)
                cvec = hcol_v[pl.ds(g * _W + w0, _LANES)]

                @pl.loop(0, D)
                def _(c):
                    vals = plsc.load_gather(
                        bufa.at[slot], [rows16, cvec + c]
                    )
                    buft[slot, c, pl.ds(w0, _LANES)] = vals

            out_copies[slot] = pltpu.async_copy(
                buft.at[slot],
                out_hbm.at[b, h, :, pl.ds(g * _W, _W)],
                sem_o,
            )

        for cp in out_copies:
            if cp is not None:
                cp.wait()

    out4 = gather_kernel(ids_bhs, table_pairs)
    return jnp.transpose(out4, (0, 3, 1, 2))
